# baseline (speedup 1.0000x reference)
import functools

import jax
import jax.numpy as jnp
from jax import lax
from jax.experimental import pallas as pl
from jax.experimental.pallas import tpu as pltpu
from jax.experimental.pallas import tpu_sc as plsc

EMB_D = 64
NUM_K = 1024
BPB = 4
HW = 1024
ROWS = BPB * HW
N_PTS = 16 * HW
LANES = 16
UNROLL = 8


def _vq_tc_block(x_ref, e_ref, idx_ref, loss_ref):
    i = pl.program_id(0)

    xc = x_ref[...]
    xt = jnp.transpose(xc, (0, 2, 1)).reshape(ROWS, EMB_D)

    e = e_ref[...]
    e2 = e * (-2.0)
    esq = jnp.sum(e * e, axis=0, keepdims=True)

    xsq = jnp.sum(xt * xt, axis=1, keepdims=True)
    ip2 = jnp.dot(xt, e2, preferred_element_type=jnp.float32)
    d = xsq + ip2 + esq

    dmin = jnp.min(d, axis=1, keepdims=True)
    kiota = lax.broadcasted_iota(jnp.int32, (ROWS, NUM_K), 1)
    idx = jnp.min(jnp.where(d == dmin, kiota, NUM_K), axis=1)

    idx_ref[...] = idx.reshape(BPB, 1, HW)

    partial = jnp.sum(dmin)

    @pl.when(i == 0)
    def _():
        loss_ref[0, 0] = 0.0

    loss_ref[0, 0] += partial


def _vq_sc_gather(e_hbm, idx_hbm, out_hbm, tab0_v, tab1_v, idx_v,
                  out0_v, out1_v, sem):
    wid = lax.axis_index("s") * 2 + lax.axis_index("c")
    c0 = wid * 2

    pltpu.sync_copy(e_hbm.at[c0], tab0_v)
    pltpu.sync_copy(e_hbm.at[c0 + 1], tab1_v)
    pltpu.sync_copy(idx_hbm, idx_v)

    def body(j, carry):
        for u in range(UNROLL):
            s = j * (LANES * UNROLL) + u * LANES
            idxv = idx_v[pl.ds(s, LANES)]
            out0_v[pl.ds(s, LANES)] = plsc.load_gather(tab0_v, [idxv])
            out1_v[pl.ds(s, LANES)] = plsc.load_gather(tab1_v, [idxv])
        return carry

    lax.fori_loop(0, N_PTS // (LANES * UNROLL), body, 0)

    copies = []
    for b in range(16):
        copies.append(pltpu.async_copy(
            out0_v.at[pl.ds(b * HW, HW)], out_hbm.at[b, c0], sem))
        copies.append(pltpu.async_copy(
            out1_v.at[pl.ds(b * HW, HW)], out_hbm.at[b, c0 + 1], sem))
    for c in copies:
        c.wait()


def kernel(x, e_i_ts):
    B, C, H, W = x.shape
    n = B * H * W

    xr = x.reshape(B, C, H * W)

    idx3, loss_acc = pl.pallas_call(
        _vq_tc_block,
        grid=(B // BPB,),
        in_specs=[
            pl.BlockSpec((BPB, C, HW), lambda i: (i, 0, 0)),
            pl.BlockSpec((C, NUM_K), lambda i: (0, 0)),
        ],
        out_specs=[
            pl.BlockSpec((BPB, 1, HW), lambda i: (i, 0, 0)),
            pl.BlockSpec((1, 1), lambda i: (0, 0), memory_space=pltpu.SMEM),
        ],
        out_shape=[
            jax.ShapeDtypeStruct((B, 1, HW), jnp.int32),
            jax.ShapeDtypeStruct((1, 1), jnp.float32),
        ],
    )(xr, e_i_ts)

    idx_flat = idx3.reshape(n)

    sc_gather = functools.partial(
        pl.kernel,
        mesh=plsc.VectorSubcoreMesh(core_axis_name="c", subcore_axis_name="s"),
        out_type=jax.ShapeDtypeStruct((B, C, H * W), jnp.float32),
        scratch_types=[
            pltpu.VMEM((NUM_K,), jnp.float32),
            pltpu.VMEM((NUM_K,), jnp.float32),
            pltpu.VMEM((N_PTS,), jnp.int32),
            pltpu.VMEM((N_PTS,), jnp.float32),
            pltpu.VMEM((N_PTS,), jnp.float32),
            pltpu.SemaphoreType.DMA,
        ],
        compiler_params=pltpu.CompilerParams(needs_layout_passes=False),
    )(_vq_sc_gather)

    q_cm = sc_gather(e_i_ts, idx_flat)

    quantized_x_st = q_cm.reshape(B, C, H, W)
    loss = loss_acc[0, 0] / jnp.float32(n * C)
    encoding_indices = idx3.reshape(B, H * W)
    return (quantized_x_st, loss, loss, encoding_indices)


# speedup vs baseline: 1.0947x; 1.0947x over previous
import jax
import jax.numpy as jnp
from jax import lax
from jax.experimental import pallas as pl
from jax.experimental.pallas import tpu as pltpu

EMB_D = 64
NUM_K = 1024
BPB = 2
HW = 1024
ROWS = BPB * HW
RT = 64


def _vq_block(x_ref, e_ref, etb_ref, q_ref, idx_ref, loss_ref):
    i = pl.program_id(0)

    xc = x_ref[...]
    xt = jnp.transpose(xc, (0, 2, 1)).reshape(ROWS, EMB_D)

    e = e_ref[...]
    e2 = e * (-2.0)
    esq = jnp.sum(e * e, axis=0, keepdims=True)

    xsq = jnp.sum(xt * xt, axis=1, keepdims=True)
    ip2 = jnp.dot(xt, e2, preferred_element_type=jnp.float32)

    idx_parts = []
    for g in range(ROWS // RT):
        ip_g = ip2[g * RT:(g + 1) * RT, :]
        xsq_g = xsq[g * RT:(g + 1) * RT, :]
        d_g = xsq_g + ip_g + esq
        idx_parts.append(jnp.argmin(d_g, axis=1).astype(jnp.int32))
    idx_all = jnp.concatenate(idx_parts).reshape(ROWS, 1)

    kiota = lax.broadcasted_iota(jnp.int32, (ROWS, NUM_K), 1)
    onehot = (kiota == idx_all).astype(jnp.bfloat16)
    q = jnp.dot(onehot, etb_ref[...], preferred_element_type=jnp.float32)

    qst = (xt + (q - xt)).reshape(BPB, HW, EMB_D)
    q_ref[...] = jnp.transpose(qst, (0, 2, 1))
    idx_ref[...] = idx_all.reshape(BPB, 1, HW)

    diff = xt - q
    partial = jnp.sum(diff * diff)

    @pl.when(i == 0)
    def _():
        loss_ref[0, 0] = 0.0

    loss_ref[0, 0] += partial


def kernel(x, e_i_ts):
    B, C, H, W = x.shape
    n = B * H * W

    xr = x.reshape(B, C, H * W)
    etb = e_i_ts.T.astype(jnp.bfloat16)

    q_r, idx3, loss_acc = pl.pallas_call(
        _vq_block,
        grid=(B // BPB,),
        in_specs=[
            pl.BlockSpec((BPB, C, HW), lambda i: (i, 0, 0)),
            pl.BlockSpec((C, NUM_K), lambda i: (0, 0)),
            pl.BlockSpec((NUM_K, C), lambda i: (0, 0)),
        ],
        out_specs=[
            pl.BlockSpec((BPB, C, HW), lambda i: (i, 0, 0)),
            pl.BlockSpec((BPB, 1, HW), lambda i: (i, 0, 0)),
            pl.BlockSpec((1, 1), lambda i: (0, 0), memory_space=pltpu.SMEM),
        ],
        out_shape=[
            jax.ShapeDtypeStruct((B, C, H * W), jnp.float32),
            jax.ShapeDtypeStruct((B, 1, HW), jnp.int32),
            jax.ShapeDtypeStruct((1, 1), jnp.float32),
        ],
    )(xr, e_i_ts, etb)

    quantized_x_st = q_r.reshape(B, C, H, W)
    loss = loss_acc[0, 0] / jnp.float32(n * C)
    encoding_indices = idx3.reshape(B, H * W)
    return (quantized_x_st, loss, loss, encoding_indices)


# speedup vs baseline: 1.5573x; 1.4226x over previous
import jax
import jax.numpy as jnp
from jax.experimental import pallas as pl
from jax.experimental.pallas import tpu as pltpu

EMB_D = 64
NUM_K = 1024
BPB = 4
HW = 1024
ROWS = BPB * HW


def _vq_block(x_ref, e_ref, etb_ref, q_ref, idx_ref, loss_ref):
    i = pl.program_id(0)

    xc = x_ref[...]
    xt = jnp.transpose(xc, (0, 2, 1)).reshape(ROWS, EMB_D)

    e = e_ref[...]
    e2 = e * (-2.0)
    esq = jnp.sum(e * e, axis=0, keepdims=True)

    xsq = jnp.sum(xt * xt, axis=1, keepdims=True)
    ip2 = jnp.dot(xt, e2, preferred_element_type=jnp.float32)
    d = xsq + ip2 + esq

    idx = jnp.argmin(d, axis=1).astype(jnp.int32)

    kiota = jax.lax.broadcasted_iota(jnp.int32, (ROWS, NUM_K), 1)
    onehot = (kiota == idx[:, None]).astype(jnp.bfloat16)
    eb = e.astype(jnp.bfloat16)

    partial = jnp.float32(0.0)
    for b in range(BPB):
        oh_b = onehot[b * HW:(b + 1) * HW, :]
        q_b = jax.lax.dot_general(
            eb, oh_b,
            dimension_numbers=(((1,), (1,)), ((), ())),
            preferred_element_type=jnp.float32,
        )
        xb = xc[b]
        q_ref[b] = xb + (q_b - xb)
        diff = xb - q_b
        partial = partial + jnp.sum(diff * diff)

    idx_ref[...] = idx.reshape(BPB, 1, HW)

    @pl.when(i == 0)
    def _():
        loss_ref[0, 0] = 0.0

    loss_ref[0, 0] += partial


def kernel(x, e_i_ts):
    B, C, H, W = x.shape
    n = B * H * W

    xr = x.reshape(B, C, H * W)
    etb = e_i_ts.T.astype(jnp.bfloat16)

    q_r, idx3, loss_acc = pl.pallas_call(
        _vq_block,
        grid=(B // BPB,),
        in_specs=[
            pl.BlockSpec((BPB, C, HW), lambda i: (i, 0, 0)),
            pl.BlockSpec((C, NUM_K), lambda i: (0, 0)),
            pl.BlockSpec((NUM_K, C), lambda i: (0, 0)),
        ],
        out_specs=[
            pl.BlockSpec((BPB, C, HW), lambda i: (i, 0, 0)),
            pl.BlockSpec((BPB, 1, HW), lambda i: (i, 0, 0)),
            pl.BlockSpec((1, 1), lambda i: (0, 0), memory_space=pltpu.SMEM),
        ],
        out_shape=[
            jax.ShapeDtypeStruct((B, C, H * W), jnp.float32),
            jax.ShapeDtypeStruct((B, 1, HW), jnp.int32),
            jax.ShapeDtypeStruct((1, 1), jnp.float32),
        ],
    )(xr, e_i_ts, etb)

    quantized_x_st = q_r.reshape(B, C, H, W)
    loss = loss_acc[0, 0] / jnp.float32(n * C)
    encoding_indices = idx3.reshape(B, H * W)
    return (quantized_x_st, loss, loss, encoding_indices)
